# baseline (speedup 1.0000x reference)
import functools

import jax
import jax.numpy as jnp
from jax import lax
from jax.experimental import pallas as pl
from jax.experimental.pallas import tpu as pltpu
from jax.experimental.pallas import tpu_sc as plsc

B, S, D = 1, 2048, 1024
E, TOPK, NZ, FF, ZH = 16, 2, 8, 512, 256
T = B * S
TM = 256
R = T * TOPK + E * TM
NT = R // TM

NC, NS = 2, 16
NW = NC * NS
CHUNK = T // NW


TB = 256


def _routing1_body(x_ref, gu_ref, gw_ref, zw1_ref, zb1_ref, zw2_ref,
                   zb2_ref, u_ref, w0_ref, w1_ref, e0_ref, e1_ref, msum_ref):
    x = x_ref[...]
    rl = lax.dot_general(x, gw_ref[...], (((1,), (1,)), ((), ())),
                         preferred_element_type=jnp.float32)
    h1 = lax.dot_general(x, zw1_ref[...], (((1,), (1,)), ((), ())),
                         preferred_element_type=jnp.float32)
    h1 = h1 + zb1_ref[...]
    h1 = h1 * jax.nn.sigmoid(h1)
    zl = lax.dot_general(h1, zw2_ref[...], (((1,), (1,)), ((), ())),
                         preferred_element_type=jnp.float32)
    zl = zl + zb2_ref[...]
    g = -jnp.log(-jnp.log(gu_ref[...]))
    s = zl + g
    iota_nz = lax.broadcasted_iota(jnp.int32, (TB, NZ), 1)
    smax = jnp.max(s, axis=1, keepdims=True)
    zidx = jnp.min(jnp.where(s >= smax, iota_nz, NZ), axis=1, keepdims=True)
    oh = jnp.where(iota_nz == zidx, 1.0, 0.0).astype(jnp.float32)
    bias = jnp.dot(oh, u_ref[...], preferred_element_type=jnp.float32)
    logits = rl + bias
    lmax = jnp.max(logits, axis=1, keepdims=True)
    ex = jnp.exp(logits - lmax)
    rw = ex / jnp.sum(ex, axis=1, keepdims=True)
    iota_e = lax.broadcasted_iota(jnp.int32, (TB, E), 1)
    w0 = jnp.max(rw, axis=1, keepdims=True)
    e0 = jnp.min(jnp.where(rw >= w0, iota_e, E), axis=1, keepdims=True)
    rw2 = jnp.where(iota_e == e0, -1.0, rw)
    w1 = jnp.max(rw2, axis=1, keepdims=True)
    e1 = jnp.min(jnp.where(rw2 >= w1, iota_e, E), axis=1, keepdims=True)
    w0_ref[...] = w0
    w1_ref[...] = w1
    e0_ref[...] = e0
    e1_ref[...] = e1
    msum_ref[...] = ((iota_e == e0).astype(jnp.float32) +
                     (iota_e == e1).astype(jnp.float32))


def _routing1(x, gumbel_u, gate_weight, zW1, zb1, zW2, zb2, U):
    nb = T // TB
    return pl.pallas_call(
        _routing1_body,
        grid=(nb,),
        in_specs=[
            pl.BlockSpec((TB, D), lambda i: (i, 0)),
            pl.BlockSpec((TB, NZ), lambda i: (i, 0)),
            pl.BlockSpec((E, D), lambda i: (0, 0)),
            pl.BlockSpec((ZH, D), lambda i: (0, 0)),
            pl.BlockSpec((1, ZH), lambda i: (0, 0)),
            pl.BlockSpec((NZ, ZH), lambda i: (0, 0)),
            pl.BlockSpec((1, NZ), lambda i: (0, 0)),
            pl.BlockSpec((NZ, E), lambda i: (0, 0)),
        ],
        out_specs=(
            pl.BlockSpec((TB, 1), lambda i: (i, 0)),
            pl.BlockSpec((TB, 1), lambda i: (i, 0)),
            pl.BlockSpec((TB, 1), lambda i: (i, 0)),
            pl.BlockSpec((TB, 1), lambda i: (i, 0)),
            pl.BlockSpec((TB, E), lambda i: (i, 0)),
        ),
        out_shape=(
            jax.ShapeDtypeStruct((T, 1), jnp.float32),
            jax.ShapeDtypeStruct((T, 1), jnp.float32),
            jax.ShapeDtypeStruct((T, 1), jnp.int32),
            jax.ShapeDtypeStruct((T, 1), jnp.int32),
            jax.ShapeDtypeStruct((T, E), jnp.float32),
        ),
    )(x, gumbel_u, gate_weight, zW1, zb1.reshape(1, ZH), zW2,
      zb2.reshape(1, NZ), U)


def _routing2_body(msum_ref, e0_ref, e1_ref, row0_ref, row1_ref, toffs_ref):
    msum = msum_ref[...]
    e0 = e0_ref[...]
    e1 = e1_ref[...]
    iota_e = lax.broadcasted_iota(jnp.int32, (T, E), 1)
    ir = lax.broadcasted_iota(jnp.int32, (T, T), 0)
    ic = lax.broadcasted_iota(jnp.int32, (T, T), 1)
    tri = (ic < ir).astype(jnp.bfloat16)
    excl = jnp.dot(tri, msum.astype(jnp.bfloat16),
                   preferred_element_type=jnp.float32)
    excl_i = excl.astype(jnp.int32)
    cnt = jnp.sum(msum, axis=0, keepdims=True).astype(jnp.int32)
    padded = ((cnt + (TM - 1)) // TM) * TM
    ier = lax.broadcasted_iota(jnp.int32, (E, E), 0)
    iec = lax.broadcasted_iota(jnp.int32, (E, E), 1)
    pad_col = jnp.broadcast_to(padded.reshape(E, 1), (E, E))
    offs = jnp.sum(jnp.where(ier < iec, pad_col, 0), axis=0,
                   keepdims=True)
    row0 = jnp.sum(jnp.where(iota_e == e0, offs + excl_i, 0), axis=1,
                   keepdims=True)
    row1 = jnp.sum(jnp.where(iota_e == e1, offs + excl_i, 0), axis=1,
                   keepdims=True)
    row0_ref[...] = row0
    row1_ref[...] = row1

    ir17 = lax.broadcasted_iota(jnp.int32, (E + 1, E), 0)
    ic17 = lax.broadcasted_iota(jnp.int32, (E + 1, E), 1)
    pad17 = jnp.broadcast_to(padded, (E + 1, E))
    toffs_ref[...] = (jnp.sum(jnp.where(ic17 < ir17, pad17, 0), axis=1,
                              keepdims=True) // TM)


def _routing2(msum, e0, e1):
    return pl.pallas_call(
        _routing2_body,
        out_shape=(
            jax.ShapeDtypeStruct((T, 1), jnp.int32),
            jax.ShapeDtypeStruct((T, 1), jnp.int32),
            jax.ShapeDtypeStruct((E + 1, 1), jnp.int32),
        ),
    )(msum, e0, e1)


def _make_sc_scatter():
    mesh = plsc.VectorSubcoreMesh(core_axis_name="c", subcore_axis_name="s")

    @functools.partial(
        pl.kernel, mesh=mesh,
        out_type=jax.ShapeDtypeStruct((R, D), jnp.float32),
        scratch_types=[
            pltpu.VMEM((CHUNK,), jnp.int32),
            pltpu.VMEM((CHUNK,), jnp.int32),
            pltpu.VMEM((CHUNK, D), jnp.float32),
            pltpu.SemaphoreType.DMA,
            pltpu.SemaphoreType.DMA,
        ],
    )
    def sc_scatter(x_hbm, r0_hbm, r1_hbm, xs_hbm, idx0_v, idx1_v, rows_v,
                   sem0, sem1):
        wid = lax.axis_index("s") * NC + lax.axis_index("c")
        base = wid * CHUNK
        pltpu.sync_copy(r0_hbm.at[pl.ds(base, CHUNK)], idx0_v)
        pltpu.sync_copy(r1_hbm.at[pl.ds(base, CHUNK)], idx1_v)
        pltpu.sync_copy(x_hbm.at[pl.ds(base, CHUNK)], rows_v)
        d0 = pltpu.async_copy(rows_v, xs_hbm.at[idx0_v], sem0)
        d1 = pltpu.async_copy(rows_v, xs_hbm.at[idx1_v], sem1)
        d0.wait()
        d1.wait()

    return sc_scatter


@functools.cache
def _get_sc_scatter():
    return _make_sc_scatter()


def _sc_scatter(x, r0, r1):
    return _get_sc_scatter()(x, r0, r1)


def _make_sc_gather():
    mesh = plsc.VectorSubcoreMesh(core_axis_name="c", subcore_axis_name="s")

    @functools.partial(
        pl.kernel, mesh=mesh,
        out_type=(jax.ShapeDtypeStruct((T, D), jnp.float32),
                  jax.ShapeDtypeStruct((T, D), jnp.float32)),
        scratch_types=[
            pltpu.VMEM((CHUNK,), jnp.int32),
            pltpu.VMEM((CHUNK, D), jnp.float32),
            pltpu.SemaphoreType.DMA,
        ],
    )
    def sc_gather(ys_hbm, r0_hbm, r1_hbm, y0_hbm, y1_hbm, idx_v, rows_v, sem):
        wid = lax.axis_index("s") * NC + lax.axis_index("c")
        base = wid * CHUNK
        pltpu.sync_copy(r0_hbm.at[pl.ds(base, CHUNK)], idx_v)
        pltpu.async_copy(ys_hbm.at[idx_v], rows_v, sem).wait()
        pltpu.sync_copy(rows_v, y0_hbm.at[pl.ds(base, CHUNK)])
        pltpu.sync_copy(r1_hbm.at[pl.ds(base, CHUNK)], idx_v)
        pltpu.async_copy(ys_hbm.at[idx_v], rows_v, sem).wait()
        pltpu.sync_copy(rows_v, y1_hbm.at[pl.ds(base, CHUNK)])

    return sc_gather


@functools.cache
def _get_sc_gather():
    return _make_sc_gather()


def _sc_gather(ys, r0, r1):
    return _get_sc_gather()(ys, r0, r1)


def _experts_body(toffs_ref, xs_hbm, wg_ref, wu_ref, wd_ref, ys_hbm,
                  xst, yst, wgub, wdb, sin, sout):
    e = pl.program_id(0)
    wgub[:FF] = wg_ref[0].astype(jnp.bfloat16)
    wgub[FF:] = wu_ref[0].astype(jnp.bfloat16)
    wdb[...] = wd_ref[0].astype(jnp.bfloat16)

    lo = toffs_ref[e]
    hi = toffs_ref[e + 1]
    n = hi - lo

    def start_in(t, slot):
        pltpu.make_async_copy(xs_hbm.at[pl.ds(t * TM, TM)], xst.at[slot],
                              sin.at[slot]).start()

    @pl.when(n > 0)
    def _():
        start_in(lo, lax.rem(lo, 2))

    def tile(t, carry):
        slot = lax.rem(t, 2)
        nslot = lax.rem(t + 1, 2)

        @pl.when(t + 1 < hi)
        def _():
            start_in(t + 1, nslot)

        pltpu.make_async_copy(xs_hbm.at[pl.ds(t * TM, TM)], xst.at[slot],
                              sin.at[slot]).wait()
        xb = xst[slot].astype(jnp.bfloat16)
        gu = lax.dot_general(xb, wgub[...], (((1,), (1,)), ((), ())),
                             preferred_element_type=jnp.float32)
        gp = gu[:, :FF]
        up = gu[:, FF:]
        hm = (gp * jax.nn.sigmoid(gp) * up).astype(jnp.bfloat16)
        y = lax.dot_general(hm, wdb[...], (((1,), (1,)), ((), ())),
                            preferred_element_type=jnp.float32)

        @pl.when(t - 2 >= lo)
        def _():
            pltpu.make_async_copy(yst.at[slot],
                                  ys_hbm.at[pl.ds((t - 2) * TM, TM)],
                                  sout.at[slot]).wait()

        yst[slot] = y
        pltpu.make_async_copy(yst.at[slot], ys_hbm.at[pl.ds(t * TM, TM)],
                              sout.at[slot]).start()
        return carry

    lax.fori_loop(lo, hi, tile, 0)

    @pl.when(n > 0)
    def _():
        pltpu.make_async_copy(yst.at[lax.rem(hi - 1, 2)],
                              ys_hbm.at[pl.ds((hi - 1) * TM, TM)],
                              sout.at[lax.rem(hi - 1, 2)]).wait()

    @pl.when(n > 1)
    def _():
        pltpu.make_async_copy(yst.at[lax.rem(hi - 2, 2)],
                              ys_hbm.at[pl.ds((hi - 2) * TM, TM)],
                              sout.at[lax.rem(hi - 2, 2)]).wait()


def _experts(xs, toffs, Wg, Wu, Wd):
    grid_spec = pltpu.PrefetchScalarGridSpec(
        num_scalar_prefetch=1,
        grid=(E,),
        in_specs=[
            pl.BlockSpec(memory_space=pltpu.MemorySpace.HBM),
            pl.BlockSpec((1, FF, D), lambda e, toffs: (e, 0, 0)),
            pl.BlockSpec((1, FF, D), lambda e, toffs: (e, 0, 0)),
            pl.BlockSpec((1, D, FF), lambda e, toffs: (e, 0, 0)),
        ],
        out_specs=pl.BlockSpec(memory_space=pltpu.MemorySpace.HBM),
        scratch_shapes=[
            pltpu.VMEM((2, TM, D), jnp.float32),
            pltpu.VMEM((2, TM, D), jnp.float32),
            pltpu.VMEM((2 * FF, D), jnp.bfloat16),
            pltpu.VMEM((D, FF), jnp.bfloat16),
            pltpu.SemaphoreType.DMA((2,)),
            pltpu.SemaphoreType.DMA((2,)),
        ],
    )
    return pl.pallas_call(
        _experts_body,
        grid_spec=grid_spec,
        out_shape=jax.ShapeDtypeStruct((R, D), jnp.float32),
    )(toffs, xs, Wg, Wu, Wd)


def _combine_body(y0_ref, y1_ref, w0_ref, w1_ref, out_ref):
    out_ref[...] = w0_ref[...] * y0_ref[...] + w1_ref[...] * y1_ref[...]


def _combine(y0, y1, w0, w1):
    nb = 8
    tb = T // nb
    return pl.pallas_call(
        _combine_body,
        grid=(nb,),
        in_specs=[
            pl.BlockSpec((tb, D), lambda i: (i, 0)),
            pl.BlockSpec((tb, D), lambda i: (i, 0)),
            pl.BlockSpec((tb, 1), lambda i: (i, 0)),
            pl.BlockSpec((tb, 1), lambda i: (i, 0)),
        ],
        out_specs=pl.BlockSpec((tb, D), lambda i: (i, 0)),
        out_shape=jax.ShapeDtypeStruct((T, D), jnp.float32),
    )(y0, y1, w0, w1)


def kernel(hidden_states, gumbel_u, gate_weight, zW1, zb1, zW2, zb2, U, Wg,
           Wu, Wd):
    x = hidden_states.reshape(T, D)
    w0, w1, e0, e1, msum = _routing1(x, gumbel_u, gate_weight, zW1, zb1,
                                     zW2, zb2, U)
    row0, row1, toffs = _routing2(msum, e0, e1)
    r0 = row0.reshape(T)
    r1 = row1.reshape(T)
    xs = _sc_scatter(x, r0, r1)
    ys = _experts(xs, toffs.reshape(E + 1), Wg, Wu, Wd)
    y0, y1 = _sc_gather(ys, r0, r1)
    out = _combine(y0, y1, w0, w1)
    return out.reshape(B, S, D)


# speedup vs baseline: 1.0429x; 1.0429x over previous
import functools

import jax
import jax.numpy as jnp
from jax import lax
from jax.experimental import pallas as pl
from jax.experimental.pallas import tpu as pltpu
from jax.experimental.pallas import tpu_sc as plsc

B, S, D = 1, 2048, 1024
E, TOPK, NZ, FF, ZH = 16, 2, 8, 512, 256
T = B * S
TM = 256
R = T * TOPK + E * TM
NT = R // TM

NC, NS = 2, 16
NW = NC * NS
CHUNK = T // NW


def _routing_body(x_ref, gu_ref, gw_ref, zw1_ref, zb1_ref, zw2_ref, zb2_ref,
                  u_ref, w0_ref, w1_ref, row0_ref, row1_ref, toffs_ref):
    x = x_ref[...]
    rl = lax.dot_general(x, gw_ref[...], (((1,), (1,)), ((), ())),
                         preferred_element_type=jnp.float32)
    h1 = lax.dot_general(x, zw1_ref[...], (((1,), (1,)), ((), ())),
                         preferred_element_type=jnp.float32)
    h1 = h1 + zb1_ref[...]
    h1 = h1 * jax.nn.sigmoid(h1)
    zl = lax.dot_general(h1, zw2_ref[...], (((1,), (1,)), ((), ())),
                         preferred_element_type=jnp.float32)
    zl = zl + zb2_ref[...]
    g = -jnp.log(-jnp.log(gu_ref[...]))
    s = zl + g
    iota_nz = lax.broadcasted_iota(jnp.int32, (T, NZ), 1)
    smax = jnp.max(s, axis=1, keepdims=True)
    zidx = jnp.min(jnp.where(s >= smax, iota_nz, NZ), axis=1, keepdims=True)
    oh = jnp.where(iota_nz == zidx, 1.0, 0.0).astype(jnp.float32)
    bias = jnp.dot(oh, u_ref[...], preferred_element_type=jnp.float32)
    logits = rl + bias
    lmax = jnp.max(logits, axis=1, keepdims=True)
    ex = jnp.exp(logits - lmax)
    rw = ex / jnp.sum(ex, axis=1, keepdims=True)
    iota_e = lax.broadcasted_iota(jnp.int32, (T, E), 1)
    w0 = jnp.max(rw, axis=1, keepdims=True)
    e0 = jnp.min(jnp.where(rw >= w0, iota_e, E), axis=1, keepdims=True)
    rw2 = jnp.where(iota_e == e0, -1.0, rw)
    w1 = jnp.max(rw2, axis=1, keepdims=True)
    e1 = jnp.min(jnp.where(rw2 >= w1, iota_e, E), axis=1, keepdims=True)
    w0_ref[...] = w0
    w1_ref[...] = w1

    m0 = (iota_e == e0).astype(jnp.float32)
    m1 = (iota_e == e1).astype(jnp.float32)
    msum = m0 + m1
    ir = lax.broadcasted_iota(jnp.int32, (T, T), 0)
    ic = lax.broadcasted_iota(jnp.int32, (T, T), 1)
    tri = (ic < ir).astype(jnp.bfloat16)
    excl = jnp.dot(tri, msum.astype(jnp.bfloat16),
                   preferred_element_type=jnp.float32)
    excl_i = excl.astype(jnp.int32)
    cnt = jnp.sum(msum, axis=0, keepdims=True).astype(jnp.int32)
    padded = ((cnt + (TM - 1)) // TM) * TM
    ier = lax.broadcasted_iota(jnp.int32, (E, E), 0)
    iec = lax.broadcasted_iota(jnp.int32, (E, E), 1)
    pad_col = jnp.broadcast_to(padded.reshape(E, 1), (E, E))
    offs = jnp.sum(jnp.where(ier < iec, pad_col, 0), axis=0,
                   keepdims=True)
    row0 = jnp.sum(jnp.where(iota_e == e0, offs + excl_i, 0), axis=1,
                   keepdims=True)
    row1 = jnp.sum(jnp.where(iota_e == e1, offs + excl_i, 0), axis=1,
                   keepdims=True)
    row0_ref[...] = row0
    row1_ref[...] = row1

    ir17 = lax.broadcasted_iota(jnp.int32, (E + 1, E), 0)
    ic17 = lax.broadcasted_iota(jnp.int32, (E + 1, E), 1)
    pad17 = jnp.broadcast_to(padded, (E + 1, E))
    toffs_ref[...] = (jnp.sum(jnp.where(ic17 < ir17, pad17, 0), axis=1,
                              keepdims=True) // TM)


def _routing(x, gumbel_u, gate_weight, zW1, zb1, zW2, zb2, U):
    return pl.pallas_call(
        _routing_body,
        out_shape=(
            jax.ShapeDtypeStruct((T, 1), jnp.float32),
            jax.ShapeDtypeStruct((T, 1), jnp.float32),
            jax.ShapeDtypeStruct((T, 1), jnp.int32),
            jax.ShapeDtypeStruct((T, 1), jnp.int32),
            jax.ShapeDtypeStruct((E + 1, 1), jnp.int32),
        ),
    )(x, gumbel_u, gate_weight, zW1, zb1.reshape(1, ZH), zW2,
      zb2.reshape(1, NZ), U)


def _make_sc_scatter():
    mesh = plsc.VectorSubcoreMesh(core_axis_name="c", subcore_axis_name="s")

    @functools.partial(
        pl.kernel, mesh=mesh,
        out_type=jax.ShapeDtypeStruct((R, D), jnp.float32),
        scratch_types=[
            pltpu.VMEM((CHUNK,), jnp.int32),
            pltpu.VMEM((CHUNK,), jnp.int32),
            pltpu.VMEM((CHUNK, D), jnp.float32),
            pltpu.SemaphoreType.DMA,
            pltpu.SemaphoreType.DMA,
        ],
    )
    def sc_scatter(x_hbm, r0_hbm, r1_hbm, xs_hbm, idx0_v, idx1_v, rows_v,
                   sem0, sem1):
        wid = lax.axis_index("s") * NC + lax.axis_index("c")
        base = wid * CHUNK
        pltpu.sync_copy(r0_hbm.at[pl.ds(base, CHUNK)], idx0_v)
        pltpu.sync_copy(r1_hbm.at[pl.ds(base, CHUNK)], idx1_v)
        pltpu.sync_copy(x_hbm.at[pl.ds(base, CHUNK)], rows_v)
        d0 = pltpu.async_copy(rows_v, xs_hbm.at[idx0_v], sem0)
        d1 = pltpu.async_copy(rows_v, xs_hbm.at[idx1_v], sem1)
        d0.wait()
        d1.wait()

    return sc_scatter


@functools.cache
def _get_sc_scatter():
    return _make_sc_scatter()


def _sc_scatter(x, r0, r1):
    return _get_sc_scatter()(x, r0, r1)


def _make_sc_gather():
    mesh = plsc.VectorSubcoreMesh(core_axis_name="c", subcore_axis_name="s")

    @functools.partial(
        pl.kernel, mesh=mesh,
        out_type=(jax.ShapeDtypeStruct((T, D), jnp.float32),
                  jax.ShapeDtypeStruct((T, D), jnp.float32)),
        scratch_types=[
            pltpu.VMEM((CHUNK,), jnp.int32),
            pltpu.VMEM((CHUNK, D), jnp.float32),
            pltpu.SemaphoreType.DMA,
        ],
    )
    def sc_gather(ys_hbm, r0_hbm, r1_hbm, y0_hbm, y1_hbm, idx_v, rows_v, sem):
        wid = lax.axis_index("s") * NC + lax.axis_index("c")
        base = wid * CHUNK
        pltpu.sync_copy(r0_hbm.at[pl.ds(base, CHUNK)], idx_v)
        pltpu.async_copy(ys_hbm.at[idx_v], rows_v, sem).wait()
        pltpu.sync_copy(rows_v, y0_hbm.at[pl.ds(base, CHUNK)])
        pltpu.sync_copy(r1_hbm.at[pl.ds(base, CHUNK)], idx_v)
        pltpu.async_copy(ys_hbm.at[idx_v], rows_v, sem).wait()
        pltpu.sync_copy(rows_v, y1_hbm.at[pl.ds(base, CHUNK)])

    return sc_gather


@functools.cache
def _get_sc_gather():
    return _make_sc_gather()


def _sc_gather(ys, r0, r1):
    return _get_sc_gather()(ys, r0, r1)


def _experts_body(toffs_ref, xs_hbm, wg_ref, wu_ref, wd_ref, ys_hbm,
                  xst, yst, wgub, wdb, sin, sout):
    e = pl.program_id(0)
    wgub[:FF] = wg_ref[0].astype(jnp.bfloat16)
    wgub[FF:] = wu_ref[0].astype(jnp.bfloat16)
    wdb[...] = wd_ref[0].astype(jnp.bfloat16)

    lo = toffs_ref[e]
    hi = toffs_ref[e + 1]
    n = hi - lo

    def start_in(t, slot):
        pltpu.make_async_copy(xs_hbm.at[pl.ds(t * TM, TM)], xst.at[slot],
                              sin.at[slot]).start()

    @pl.when(n > 0)
    def _():
        start_in(lo, lax.rem(lo, 2))

    def tile(t, carry):
        slot = lax.rem(t, 2)
        nslot = lax.rem(t + 1, 2)

        @pl.when(t + 1 < hi)
        def _():
            start_in(t + 1, nslot)

        pltpu.make_async_copy(xs_hbm.at[pl.ds(t * TM, TM)], xst.at[slot],
                              sin.at[slot]).wait()
        xb = xst[slot].astype(jnp.bfloat16)
        gu = lax.dot_general(xb, wgub[...], (((1,), (1,)), ((), ())),
                             preferred_element_type=jnp.float32)
        gp = gu[:, :FF]
        up = gu[:, FF:]
        hm = (gp * jax.nn.sigmoid(gp) * up).astype(jnp.bfloat16)
        y = lax.dot_general(hm, wdb[...], (((1,), (1,)), ((), ())),
                            preferred_element_type=jnp.float32)

        @pl.when(t - 2 >= lo)
        def _():
            pltpu.make_async_copy(yst.at[slot],
                                  ys_hbm.at[pl.ds((t - 2) * TM, TM)],
                                  sout.at[slot]).wait()

        yst[slot] = y
        pltpu.make_async_copy(yst.at[slot], ys_hbm.at[pl.ds(t * TM, TM)],
                              sout.at[slot]).start()
        return carry

    lax.fori_loop(lo, hi, tile, 0)

    @pl.when(n > 0)
    def _():
        pltpu.make_async_copy(yst.at[lax.rem(hi - 1, 2)],
                              ys_hbm.at[pl.ds((hi - 1) * TM, TM)],
                              sout.at[lax.rem(hi - 1, 2)]).wait()

    @pl.when(n > 1)
    def _():
        pltpu.make_async_copy(yst.at[lax.rem(hi - 2, 2)],
                              ys_hbm.at[pl.ds((hi - 2) * TM, TM)],
                              sout.at[lax.rem(hi - 2, 2)]).wait()


def _experts(xs, toffs, Wg, Wu, Wd):
    grid_spec = pltpu.PrefetchScalarGridSpec(
        num_scalar_prefetch=1,
        grid=(E,),
        in_specs=[
            pl.BlockSpec(memory_space=pltpu.MemorySpace.HBM),
            pl.BlockSpec((1, FF, D), lambda e, toffs: (e, 0, 0)),
            pl.BlockSpec((1, FF, D), lambda e, toffs: (e, 0, 0)),
            pl.BlockSpec((1, D, FF), lambda e, toffs: (e, 0, 0)),
        ],
        out_specs=pl.BlockSpec(memory_space=pltpu.MemorySpace.HBM),
        scratch_shapes=[
            pltpu.VMEM((2, TM, D), jnp.float32),
            pltpu.VMEM((2, TM, D), jnp.float32),
            pltpu.VMEM((2 * FF, D), jnp.bfloat16),
            pltpu.VMEM((D, FF), jnp.bfloat16),
            pltpu.SemaphoreType.DMA((2,)),
            pltpu.SemaphoreType.DMA((2,)),
        ],
    )
    return pl.pallas_call(
        _experts_body,
        grid_spec=grid_spec,
        out_shape=jax.ShapeDtypeStruct((R, D), jnp.float32),
    )(toffs, xs, Wg, Wu, Wd)


def _combine_body(y0_ref, y1_ref, w0_ref, w1_ref, out_ref):
    out_ref[...] = w0_ref[...] * y0_ref[...] + w1_ref[...] * y1_ref[...]


def _combine(y0, y1, w0, w1):
    nb = 8
    tb = T // nb
    return pl.pallas_call(
        _combine_body,
        grid=(nb,),
        in_specs=[
            pl.BlockSpec((tb, D), lambda i: (i, 0)),
            pl.BlockSpec((tb, D), lambda i: (i, 0)),
            pl.BlockSpec((tb, 1), lambda i: (i, 0)),
            pl.BlockSpec((tb, 1), lambda i: (i, 0)),
        ],
        out_specs=pl.BlockSpec((tb, D), lambda i: (i, 0)),
        out_shape=jax.ShapeDtypeStruct((T, D), jnp.float32),
    )(y0, y1, w0, w1)


def kernel(hidden_states, gumbel_u, gate_weight, zW1, zb1, zW2, zb2, U, Wg,
           Wu, Wd):
    x = hidden_states.reshape(T, D)
    w0, w1, row0, row1, toffs = _routing(x, gumbel_u, gate_weight, zW1, zb1,
                                         zW2, zb2, U)
    r0 = row0.reshape(T)
    r1 = row1.reshape(T)
    xs = _sc_scatter(x, r0, r1)
    ys = _experts(xs, toffs.reshape(E + 1), Wg, Wu, Wd)
    y0, y1 = _sc_gather(ys, r0, r1)
    out = _combine(y0, y1, w0, w1)
    return out.reshape(B, S, D)


# speedup vs baseline: 1.0488x; 1.0056x over previous
import functools

import jax
import jax.numpy as jnp
from jax import lax
from jax.experimental import pallas as pl
from jax.experimental.pallas import tpu as pltpu
from jax.experimental.pallas import tpu_sc as plsc

B, S, D = 1, 2048, 1024
E, TOPK, NZ, FF, ZH = 16, 2, 8, 512, 256
T = B * S
TM = 256
R = T * TOPK + E * TM

NC, NS = 2, 16
NW = NC * NS
CHUNK = T // NW


def _routing_body(x_ref, gu_ref, gw_ref, zw1_ref, zb1_ref, zw2_ref, zb2_ref,
                  u_ref, w0_ref, w1_ref, row0_ref, row1_ref, toffs_ref):
    x = x_ref[...]
    rl = lax.dot_general(x, gw_ref[...], (((1,), (1,)), ((), ())),
                         preferred_element_type=jnp.float32)
    h1 = lax.dot_general(x, zw1_ref[...], (((1,), (1,)), ((), ())),
                         preferred_element_type=jnp.float32)
    h1 = h1 + zb1_ref[...]
    h1 = h1 * jax.nn.sigmoid(h1)
    zl = lax.dot_general(h1, zw2_ref[...], (((1,), (1,)), ((), ())),
                         preferred_element_type=jnp.float32)
    zl = zl + zb2_ref[...]
    g = -jnp.log(-jnp.log(gu_ref[...]))
    s = zl + g
    iota_nz = lax.broadcasted_iota(jnp.int32, (T, NZ), 1)
    smax = jnp.max(s, axis=1, keepdims=True)
    zidx = jnp.min(jnp.where(s >= smax, iota_nz, NZ), axis=1, keepdims=True)
    oh = jnp.where(iota_nz == zidx, 1.0, 0.0).astype(jnp.float32)
    bias = jnp.dot(oh, u_ref[...], preferred_element_type=jnp.float32)
    logits = rl + bias
    lmax = jnp.max(logits, axis=1, keepdims=True)
    ex = jnp.exp(logits - lmax)
    rw = ex / jnp.sum(ex, axis=1, keepdims=True)
    iota_e = lax.broadcasted_iota(jnp.int32, (T, E), 1)
    w0 = jnp.max(rw, axis=1, keepdims=True)
    e0 = jnp.min(jnp.where(rw >= w0, iota_e, E), axis=1, keepdims=True)
    rw2 = jnp.where(iota_e == e0, -1.0, rw)
    w1 = jnp.max(rw2, axis=1, keepdims=True)
    e1 = jnp.min(jnp.where(rw2 >= w1, iota_e, E), axis=1, keepdims=True)
    w0_ref[...] = w0
    w1_ref[...] = w1

    m0 = (iota_e == e0).astype(jnp.float32)
    m1 = (iota_e == e1).astype(jnp.float32)
    msum = m0 + m1
    ir = lax.broadcasted_iota(jnp.int32, (T, T), 0)
    ic = lax.broadcasted_iota(jnp.int32, (T, T), 1)
    tri = (ic < ir).astype(jnp.bfloat16)
    excl = jnp.dot(tri, msum.astype(jnp.bfloat16),
                   preferred_element_type=jnp.float32)
    excl_i = excl.astype(jnp.int32)
    cnt = jnp.sum(msum, axis=0, keepdims=True).astype(jnp.int32)
    padded = ((cnt + (TM - 1)) // TM) * TM
    ier = lax.broadcasted_iota(jnp.int32, (E, E), 0)
    iec = lax.broadcasted_iota(jnp.int32, (E, E), 1)
    pad_col = jnp.broadcast_to(padded.reshape(E, 1), (E, E))
    offs = jnp.sum(jnp.where(ier < iec, pad_col, 0), axis=0,
                   keepdims=True)
    row0 = jnp.sum(jnp.where(iota_e == e0, offs + excl_i, 0), axis=1,
                   keepdims=True)
    row1 = jnp.sum(jnp.where(iota_e == e1, offs + excl_i, 0), axis=1,
                   keepdims=True)
    row0_ref[...] = row0
    row1_ref[...] = row1

    ir17 = lax.broadcasted_iota(jnp.int32, (E + 1, E), 0)
    ic17 = lax.broadcasted_iota(jnp.int32, (E + 1, E), 1)
    pad17 = jnp.broadcast_to(padded, (E + 1, E))
    toffs_ref[...] = (jnp.sum(jnp.where(ic17 < ir17, pad17, 0), axis=1,
                              keepdims=True) // TM)


def _routing(x, gumbel_u, gate_weight, zW1, zb1, zW2, zb2, U):
    return pl.pallas_call(
        _routing_body,
        out_shape=(
            jax.ShapeDtypeStruct((T, 1), jnp.float32),
            jax.ShapeDtypeStruct((T, 1), jnp.float32),
            jax.ShapeDtypeStruct((T, 1), jnp.int32),
            jax.ShapeDtypeStruct((T, 1), jnp.int32),
            jax.ShapeDtypeStruct((E + 1, 1), jnp.int32),
        ),
    )(x, gumbel_u, gate_weight, zW1, zb1.reshape(1, ZH), zW2,
      zb2.reshape(1, NZ), U)


def _make_sc_scatter():
    mesh = plsc.VectorSubcoreMesh(core_axis_name="c", subcore_axis_name="s")

    @functools.partial(
        pl.kernel, mesh=mesh,
        out_type=jax.ShapeDtypeStruct((R, D), jnp.float32),
        scratch_types=[
            pltpu.VMEM((CHUNK,), jnp.int32),
            pltpu.VMEM((CHUNK,), jnp.int32),
            pltpu.VMEM((CHUNK, D), jnp.float32),
            pltpu.SemaphoreType.DMA,
            pltpu.SemaphoreType.DMA,
        ],
    )
    def sc_scatter(x_hbm, r0_hbm, r1_hbm, xs_hbm, idx0_v, idx1_v, rows_v,
                   sem0, sem1):
        wid = lax.axis_index("s") * NC + lax.axis_index("c")
        base = wid * CHUNK
        pltpu.sync_copy(r0_hbm.at[pl.ds(base, CHUNK)], idx0_v)
        pltpu.sync_copy(r1_hbm.at[pl.ds(base, CHUNK)], idx1_v)
        pltpu.sync_copy(x_hbm.at[pl.ds(base, CHUNK)], rows_v)
        d0 = pltpu.async_copy(rows_v, xs_hbm.at[idx0_v], sem0)
        d1 = pltpu.async_copy(rows_v, xs_hbm.at[idx1_v], sem1)
        d0.wait()
        d1.wait()

    return sc_scatter


@functools.cache
def _get_sc_scatter():
    return _make_sc_scatter()


def _sc_scatter(x, r0, r1):
    return _get_sc_scatter()(x, r0, r1)


def _make_sc_gather():
    mesh = plsc.VectorSubcoreMesh(core_axis_name="c", subcore_axis_name="s")

    @functools.partial(
        pl.kernel, mesh=mesh,
        out_type=(jax.ShapeDtypeStruct((T, D), jnp.float32),
                  jax.ShapeDtypeStruct((T, D), jnp.float32)),
        scratch_types=[
            pltpu.VMEM((CHUNK,), jnp.int32),
            pltpu.VMEM((CHUNK, D), jnp.float32),
            pltpu.SemaphoreType.DMA,
        ],
    )
    def sc_gather(ys_hbm, r0_hbm, r1_hbm, y0_hbm, y1_hbm, idx_v, rows_v, sem):
        wid = lax.axis_index("s") * NC + lax.axis_index("c")
        base = wid * CHUNK
        pltpu.sync_copy(r0_hbm.at[pl.ds(base, CHUNK)], idx_v)
        pltpu.async_copy(ys_hbm.at[idx_v], rows_v, sem).wait()
        pltpu.sync_copy(rows_v, y0_hbm.at[pl.ds(base, CHUNK)])
        pltpu.sync_copy(r1_hbm.at[pl.ds(base, CHUNK)], idx_v)
        pltpu.async_copy(ys_hbm.at[idx_v], rows_v, sem).wait()
        pltpu.sync_copy(rows_v, y1_hbm.at[pl.ds(base, CHUNK)])

    return sc_gather


@functools.cache
def _get_sc_gather():
    return _make_sc_gather()


def _sc_gather(ys, r0, r1):
    return _get_sc_gather()(ys, r0, r1)


def _experts_body(toffs_ref, xs_hbm, wg_ref, wu_ref, wd_ref, ys_hbm,
                  xst, yst, wgub, wdb, sin, sout):
    e = pl.program_id(0)
    wgub[:FF] = wg_ref[0].astype(jnp.bfloat16)
    wgub[FF:] = wu_ref[0].astype(jnp.bfloat16)
    wdb[...] = wd_ref[0].astype(jnp.bfloat16)

    lo = toffs_ref[e]
    hi = toffs_ref[e + 1]
    n = hi - lo

    def start_in(t, slot):
        pltpu.make_async_copy(xs_hbm.at[pl.ds(t * TM, TM)], xst.at[slot],
                              sin.at[slot]).start()

    @pl.when(n > 0)
    def _():
        start_in(lo, lax.rem(lo, 2))

    def tile(t, carry):
        slot = lax.rem(t, 2)
        nslot = lax.rem(t + 1, 2)

        @pl.when(t + 1 < hi)
        def _():
            start_in(t + 1, nslot)

        pltpu.make_async_copy(xs_hbm.at[pl.ds(t * TM, TM)], xst.at[slot],
                              sin.at[slot]).wait()
        xb = xst[slot].astype(jnp.bfloat16)
        gu = lax.dot_general(xb, wgub[...], (((1,), (1,)), ((), ())),
                             preferred_element_type=jnp.float32)
        gp = gu[:, :FF]
        up = gu[:, FF:]
        hm = (gp * jax.nn.sigmoid(gp) * up).astype(jnp.bfloat16)
        y = lax.dot_general(hm, wdb[...], (((1,), (1,)), ((), ())),
                            preferred_element_type=jnp.float32)

        @pl.when(t - 2 >= lo)
        def _():
            pltpu.make_async_copy(yst.at[slot],
                                  ys_hbm.at[pl.ds((t - 2) * TM, TM)],
                                  sout.at[slot]).wait()

        yst[slot] = y
        pltpu.make_async_copy(yst.at[slot], ys_hbm.at[pl.ds(t * TM, TM)],
                              sout.at[slot]).start()
        return carry

    lax.fori_loop(lo, hi, tile, 0)

    @pl.when(n > 0)
    def _():
        pltpu.make_async_copy(yst.at[lax.rem(hi - 1, 2)],
                              ys_hbm.at[pl.ds((hi - 1) * TM, TM)],
                              sout.at[lax.rem(hi - 1, 2)]).wait()

    @pl.when(n > 1)
    def _():
        pltpu.make_async_copy(yst.at[lax.rem(hi - 2, 2)],
                              ys_hbm.at[pl.ds((hi - 2) * TM, TM)],
                              sout.at[lax.rem(hi - 2, 2)]).wait()


def _experts(xs, toffs, Wg, Wu, Wd):
    grid_spec = pltpu.PrefetchScalarGridSpec(
        num_scalar_prefetch=1,
        grid=(E,),
        in_specs=[
            pl.BlockSpec(memory_space=pltpu.MemorySpace.HBM),
            pl.BlockSpec((1, FF, D), lambda e, toffs: (e, 0, 0)),
            pl.BlockSpec((1, FF, D), lambda e, toffs: (e, 0, 0)),
            pl.BlockSpec((1, D, FF), lambda e, toffs: (e, 0, 0)),
        ],
        out_specs=pl.BlockSpec(memory_space=pltpu.MemorySpace.HBM),
        scratch_shapes=[
            pltpu.VMEM((2, TM, D), jnp.float32),
            pltpu.VMEM((2, TM, D), jnp.float32),
            pltpu.VMEM((2 * FF, D), jnp.bfloat16),
            pltpu.VMEM((D, FF), jnp.bfloat16),
            pltpu.SemaphoreType.DMA((2,)),
            pltpu.SemaphoreType.DMA((2,)),
        ],
    )
    return pl.pallas_call(
        _experts_body,
        grid_spec=grid_spec,
        out_shape=jax.ShapeDtypeStruct((R, D), jnp.float32),
    )(toffs, xs, Wg, Wu, Wd)


def _combine_body(y0_ref, y1_ref, w0_ref, w1_ref, out_ref):
    out_ref[...] = w0_ref[...] * y0_ref[...] + w1_ref[...] * y1_ref[...]


def _combine(y0, y1, w0, w1):
    nb = 8
    tb = T // nb
    return pl.pallas_call(
        _combine_body,
        grid=(nb,),
        in_specs=[
            pl.BlockSpec((tb, D), lambda i: (i, 0)),
            pl.BlockSpec((tb, D), lambda i: (i, 0)),
            pl.BlockSpec((tb, 1), lambda i: (i, 0)),
            pl.BlockSpec((tb, 1), lambda i: (i, 0)),
        ],
        out_specs=pl.BlockSpec((tb, D), lambda i: (i, 0)),
        out_shape=jax.ShapeDtypeStruct((T, D), jnp.float32),
    )(y0, y1, w0, w1)


def kernel(hidden_states, gumbel_u, gate_weight, zW1, zb1, zW2, zb2, U, Wg,
           Wu, Wd):
    x = hidden_states.reshape(T, D)
    w0, w1, row0, row1, toffs = _routing(x, gumbel_u, gate_weight, zW1, zb1,
                                         zW2, zb2, U)
    r0 = row0.reshape(T)
    r1 = row1.reshape(T)
    xs = _sc_scatter(x, r0, r1)
    ys = _experts(xs, toffs.reshape(E + 1), Wg, Wu, Wd)
    y0, y1 = _sc_gather(ys, r0, r1)
    out = _combine(y0, y1, w0, w1)
    return out.reshape(B, S, D)
